# baseline (speedup 1.0000x reference)
import functools

import jax
import jax.numpy as jnp
from jax import lax
from jax.experimental import pallas as pl
from jax.experimental.pallas import tpu as pltpu
from jax.experimental.pallas import tpu_sc as plsc

D = 64
B_TOTAL = 4096 * 200
NW = 32
PER_W = B_TOTAL // NW
IW = 128
CR = 8
CH = CR * IW
NCHUNK = PER_W // CH
ROWS_TOTAL = B_TOTAL // IW

_mesh = plsc.VectorSubcoreMesh(core_axis_name="c", subcore_axis_name="s")


@functools.partial(
    pl.kernel,
    mesh=_mesh,
    out_type=jax.ShapeDtypeStruct((B_TOTAL, D), jnp.float32),
    scratch_types=[
        pltpu.VMEM((CR, IW), jnp.int32),
        pltpu.VMEM((CH, D), jnp.float32),
        pltpu.SemaphoreType.DMA,
    ],
    compiler_params=pltpu.CompilerParams(use_tc_tiling_on_sc=False),
)
def _emb_gather(idx_hbm, table_hbm, out_hbm, idx_v, rows_v, sem):
    wid = lax.axis_index("s") * 2 + lax.axis_index("c")
    row_base = wid * (PER_W // IW)

    def chunk_body(ci, carry):
        r0 = row_base + ci * CR
        pltpu.sync_copy(idx_hbm.at[pl.ds(r0, CR)], idx_v)
        copies = [
            pltpu.async_copy(
                table_hbm.at[idx_v.at[j]],
                rows_v.at[pl.ds(j * IW, IW)],
                sem,
            )
            for j in range(CR)
        ]
        for c in copies:
            c.wait()
        pltpu.sync_copy(rows_v, out_hbm.at[pl.ds(r0 * IW, CH)])
        return carry

    lax.fori_loop(0, NCHUNK, chunk_body, 0)


def kernel(input_ids, weight):
    idx2d = input_ids.reshape(ROWS_TOTAL, IW).astype(jnp.int32)
    flat = _emb_gather(idx2d, weight)
    return flat.reshape(input_ids.shape[0], input_ids.shape[1], D)


# speedup vs baseline: 1.0173x; 1.0173x over previous
import functools

import jax
import jax.numpy as jnp
from jax import lax
from jax.experimental import pallas as pl
from jax.experimental.pallas import tpu as pltpu
from jax.experimental.pallas import tpu_sc as plsc

D = 64
B_TOTAL = 4096 * 200
NW = 32
PER_W = B_TOTAL // NW
IW = 128
CR = 4
CH = CR * IW
NCHUNK = PER_W // CH
NPAIR = NCHUNK // 2
IDX_ROWS = PER_W // IW
ROWS_TOTAL = B_TOTAL // IW

_mesh = plsc.VectorSubcoreMesh(core_axis_name="c", subcore_axis_name="s")


@functools.partial(
    pl.kernel,
    mesh=_mesh,
    out_type=jax.ShapeDtypeStruct((B_TOTAL, D), jnp.float32),
    scratch_types=[
        pltpu.VMEM((IDX_ROWS, IW), jnp.int32),
        pltpu.VMEM((CH, D), jnp.float32),
        pltpu.VMEM((CH, D), jnp.float32),
        pltpu.SemaphoreType.DMA,
        pltpu.SemaphoreType.DMA,
        pltpu.SemaphoreType.DMA,
        pltpu.SemaphoreType.DMA,
    ],
    compiler_params=pltpu.CompilerParams(use_tc_tiling_on_sc=False),
)
def _emb_gather(idx_hbm, table_hbm, out_hbm, idx_v, rows0, rows1,
                sg0, sg1, so0, so1):
    wid = lax.axis_index("s") * 2 + lax.axis_index("c")
    row_base = wid * IDX_ROWS
    rows = (rows0, rows1)
    sg = (sg0, sg1)
    so = (so0, so1)

    pltpu.sync_copy(idx_hbm.at[pl.ds(row_base, IDX_ROWS)], idx_v)

    def fire_gathers(ci, b):
        for j in range(CR):
            pltpu.async_copy(
                table_hbm.at[idx_v.at[ci * CR + j]],
                rows[b].at[pl.ds(j * IW, IW)],
                sg[b],
            )

    def wait_gathers(b):
        pltpu.make_async_copy(table_hbm.at[pl.ds(0, CH)], rows[b], sg[b]).wait()

    def start_store(ci, b):
        pltpu.async_copy(
            rows[b], out_hbm.at[pl.ds((row_base + ci * CR) * IW, CH)], so[b])

    def wait_store(b):
        pltpu.make_async_copy(rows[b], out_hbm.at[pl.ds(0, CH)], so[b]).wait()

    fire_gathers(0, 0)

    def pair_body(cp, carry):
        for b in range(2):
            ci = cp * 2 + b
            wait_gathers(b)
            start_store(ci, b)
            nb = 1 - b
            @pl.when(ci >= 1)
            def _():
                wait_store(nb)

            @pl.when(ci + 1 < NCHUNK)
            def _():
                fire_gathers(ci + 1, nb)

        return carry

    lax.fori_loop(0, NPAIR, pair_body, 0)
    wait_store(1)


def kernel(input_ids, weight):
    idx2d = input_ids.reshape(ROWS_TOTAL, IW).astype(jnp.int32)
    flat = _emb_gather(idx2d, weight)
    return flat.reshape(input_ids.shape[0], input_ids.shape[1], D)
